# baseline (speedup 1.0000x reference)
import functools

import jax
import jax.numpy as jnp
from jax import lax
from jax.experimental import pallas as pl
from jax.experimental.pallas import tpu as pltpu
from jax.experimental.pallas import tpu_sc as plsc

EMB = 32
BATCH = 16384
NROWS = 1000000

NC = 2
NS = 16
NW = NC * NS
BPW = BATCH // NW

TC_BLK = 65536
TC_GRID = -(-NROWS // TC_BLK)


def _tc_body(w_ref, ut_ref, it_ref, tu_ref, tm_ref):
    wu = w_ref[0, 0:EMB].reshape(1, EMB)
    wm = w_ref[0, EMB : 2 * EMB].reshape(1, EMB)
    dn = (((1,), (0,)), ((), ()))
    tu_ref[...] = lax.dot_general(wu, ut_ref[...], dn,
                                  preferred_element_type=jnp.float32)
    tm_ref[...] = lax.dot_general(wm, it_ref[...], dn,
                                  preferred_element_type=jnp.float32)


@functools.partial(
    pl.pallas_call,
    grid=(TC_GRID,),
    in_specs=[
        pl.BlockSpec((1, 2 * EMB), lambda i: (0, 0)),
        pl.BlockSpec((EMB, TC_BLK), lambda i: (0, i)),
        pl.BlockSpec((EMB, TC_BLK), lambda i: (0, i)),
    ],
    out_specs=[
        pl.BlockSpec((1, TC_BLK), lambda i: (0, i)),
        pl.BlockSpec((1, TC_BLK), lambda i: (0, i)),
    ],
    out_shape=[
        jax.ShapeDtypeStruct((1, NROWS), jnp.float32),
        jax.ShapeDtypeStruct((1, NROWS), jnp.float32),
    ],
)
def _tc_reduce(w_ref, ut_ref, it_ref, tu_ref, tm_ref):
    _tc_body(w_ref, ut_ref, it_ref, tu_ref, tm_ref)


def _sc_body(tu, tm, users, movies, bias, out,
             uidx_v, midx_v, ublk_v, mblk_v, urow_v, mrow_v, bias_v, out_v,
             sem_u, sem_m):
    wid = lax.axis_index("s") * NC + lax.axis_index("c")
    base = wid * BPW

    pltpu.sync_copy(users.at[pl.ds(base, BPW)], uidx_v)
    pltpu.sync_copy(movies.at[pl.ds(base, BPW)], midx_v)
    pltpu.sync_copy(bias, bias_v)

    def sbody(g, carry):
        o = g * 16
        ublk_v[pl.ds(o, 16)] = lax.shift_right_logical(uidx_v[pl.ds(o, 16)], 3)
        mblk_v[pl.ds(o, 16)] = lax.shift_right_logical(midx_v[pl.ds(o, 16)], 3)
        return carry

    lax.fori_loop(0, BPW // 16, sbody, jnp.int32(0))

    cp_u = pltpu.async_copy(tu.at[ublk_v], urow_v, sem_u)
    cp_m = pltpu.async_copy(tm.at[mblk_v], mrow_v, sem_m)
    cp_u.wait()
    cp_m.wait()

    bvec = bias_v[pl.ds(0, 16)]
    iot = lax.iota(jnp.int32, 16)
    seven = jnp.full((16,), 7, jnp.int32)

    def gbody(g, carry):
        o = g * 16
        rows = o + iot
        uoff = jnp.bitwise_and(uidx_v[pl.ds(o, 16)], seven)
        moff = jnp.bitwise_and(midx_v[pl.ds(o, 16)], seven)
        vu = plsc.load_gather(urow_v, [rows, uoff])
        vm = plsc.load_gather(mrow_v, [rows, moff])
        out_v[pl.ds(o, 16)] = vu + vm + bvec
        return carry

    lax.fori_loop(0, BPW // 16, gbody, jnp.int32(0))

    pltpu.sync_copy(out_v, out.at[pl.ds(base, BPW)])


@functools.partial(
    pl.kernel,
    out_type=jax.ShapeDtypeStruct((BATCH,), jnp.float32),
    mesh=plsc.VectorSubcoreMesh(core_axis_name="c", subcore_axis_name="s"),
    scratch_types=[
        pltpu.VMEM((BPW,), jnp.int32),
        pltpu.VMEM((BPW,), jnp.int32),
        pltpu.VMEM((BPW,), jnp.int32),
        pltpu.VMEM((BPW,), jnp.int32),
        pltpu.VMEM((BPW, 8), jnp.float32),
        pltpu.VMEM((BPW, 8), jnp.float32),
        pltpu.VMEM((16,), jnp.float32),
        pltpu.VMEM((BPW,), jnp.float32),
        pltpu.SemaphoreType.DMA,
        pltpu.SemaphoreType.DMA,
    ],
    compiler_params=pltpu.CompilerParams(
        use_tc_tiling_on_sc=False, needs_layout_passes=False
    ),
)
def _sc_gather(tu, tm, users, movies, bias, out,
               uidx_v, midx_v, ublk_v, mblk_v, urow_v, mrow_v, bias_v, out_v,
               sem_u, sem_m):
    _sc_body(tu, tm, users, movies, bias, out,
             uidx_v, midx_v, ublk_v, mblk_v, urow_v, mrow_v, bias_v, out_v,
             sem_u, sem_m)


def kernel(users, movies, user_table, item_table, W, b):
    users = users.astype(jnp.int32)
    movies = movies.astype(jnp.int32)
    tu, tm = _tc_reduce(W, user_table.T, item_table.T)
    tu = tu.reshape(NROWS // 8, 8)
    tm = tm.reshape(NROWS // 8, 8)
    bias = jnp.full((16,), b[0], jnp.float32)
    out = _sc_gather(tu, tm, users, movies, bias)
    return out.reshape(BATCH, 1)


# speedup vs baseline: 1.0111x; 1.0111x over previous
import functools

import jax
import jax.numpy as jnp
from jax import lax
from jax.experimental import pallas as pl
from jax.experimental.pallas import tpu as pltpu
from jax.experimental.pallas import tpu_sc as plsc

EMB = 32
BATCH = 16384
NROWS = 1000000

NC = 2
NS = 16
NW = NC * NS
BPW = BATCH // NW

TC_BLK = 16384
TC_GRID = -(-NROWS // (2 * TC_BLK))


def _tc_body(w_ref, ut0_ref, ut1_ref, it0_ref, it1_ref, tu_ref, tm_ref):
    wu = w_ref[0, 0:EMB].reshape(1, EMB)
    wm = w_ref[0, EMB : 2 * EMB].reshape(1, EMB)
    dn = (((1,), (0,)), ((), ()))
    for src, wv, dst, off in (
        (ut0_ref, wu, tu_ref, 0), (ut1_ref, wu, tu_ref, TC_BLK),
        (it0_ref, wm, tm_ref, 0), (it1_ref, wm, tm_ref, TC_BLK),
    ):
        dst[0, pl.ds(off, TC_BLK)] = lax.dot_general(
            wv, src[...], dn, preferred_element_type=jnp.float32)[0]


@functools.partial(
    pl.pallas_call,
    grid=(TC_GRID,),
    in_specs=[
        pl.BlockSpec((1, 2 * EMB), lambda i: (0, 0)),
        pl.BlockSpec((EMB, TC_BLK), lambda i: (0, 2 * i)),
        pl.BlockSpec((EMB, TC_BLK), lambda i: (0, 2 * i + 1)),
        pl.BlockSpec((EMB, TC_BLK), lambda i: (0, 2 * i)),
        pl.BlockSpec((EMB, TC_BLK), lambda i: (0, 2 * i + 1)),
    ],
    out_specs=[
        pl.BlockSpec((1, 2 * TC_BLK), lambda i: (0, i)),
        pl.BlockSpec((1, 2 * TC_BLK), lambda i: (0, i)),
    ],
    out_shape=[
        jax.ShapeDtypeStruct((1, NROWS), jnp.float32),
        jax.ShapeDtypeStruct((1, NROWS), jnp.float32),
    ],
)
def _tc_reduce(w_ref, ut0_ref, ut1_ref, it0_ref, it1_ref, tu_ref, tm_ref):
    _tc_body(w_ref, ut0_ref, ut1_ref, it0_ref, it1_ref, tu_ref, tm_ref)


def _sc_body(tu, tm, users, movies, bias, out,
             uidx_v, midx_v, ublk_v, mblk_v, urow_v, mrow_v, bias_v, out_v,
             sem_u, sem_m):
    wid = lax.axis_index("s") * NC + lax.axis_index("c")
    base = wid * BPW

    pltpu.sync_copy(users.at[pl.ds(base, BPW)], uidx_v)
    pltpu.sync_copy(movies.at[pl.ds(base, BPW)], midx_v)
    pltpu.sync_copy(bias, bias_v)

    def sbody(g, carry):
        o = g * 16
        ublk_v[pl.ds(o, 16)] = lax.shift_right_logical(uidx_v[pl.ds(o, 16)], 3)
        mblk_v[pl.ds(o, 16)] = lax.shift_right_logical(midx_v[pl.ds(o, 16)], 3)
        return carry

    lax.fori_loop(0, BPW // 16, sbody, jnp.int32(0))

    cp_u = pltpu.async_copy(tu.at[ublk_v], urow_v, sem_u)
    cp_m = pltpu.async_copy(tm.at[mblk_v], mrow_v, sem_m)
    cp_u.wait()
    cp_m.wait()

    bvec = bias_v[pl.ds(0, 16)]
    iot = lax.iota(jnp.int32, 16)
    seven = jnp.full((16,), 7, jnp.int32)

    def gbody(g, carry):
        o = g * 16
        rows = o + iot
        uoff = jnp.bitwise_and(uidx_v[pl.ds(o, 16)], seven)
        moff = jnp.bitwise_and(midx_v[pl.ds(o, 16)], seven)
        vu = plsc.load_gather(urow_v, [rows, uoff])
        vm = plsc.load_gather(mrow_v, [rows, moff])
        out_v[pl.ds(o, 16)] = vu + vm + bvec
        return carry

    lax.fori_loop(0, BPW // 16, gbody, jnp.int32(0))

    pltpu.sync_copy(out_v, out.at[pl.ds(base, BPW)])


@functools.partial(
    pl.kernel,
    out_type=jax.ShapeDtypeStruct((BATCH,), jnp.float32),
    mesh=plsc.VectorSubcoreMesh(core_axis_name="c", subcore_axis_name="s"),
    scratch_types=[
        pltpu.VMEM((BPW,), jnp.int32),
        pltpu.VMEM((BPW,), jnp.int32),
        pltpu.VMEM((BPW,), jnp.int32),
        pltpu.VMEM((BPW,), jnp.int32),
        pltpu.VMEM((BPW, 8), jnp.float32),
        pltpu.VMEM((BPW, 8), jnp.float32),
        pltpu.VMEM((16,), jnp.float32),
        pltpu.VMEM((BPW,), jnp.float32),
        pltpu.SemaphoreType.DMA,
        pltpu.SemaphoreType.DMA,
    ],
    compiler_params=pltpu.CompilerParams(
        use_tc_tiling_on_sc=False, needs_layout_passes=False
    ),
)
def _sc_gather(tu, tm, users, movies, bias, out,
               uidx_v, midx_v, ublk_v, mblk_v, urow_v, mrow_v, bias_v, out_v,
               sem_u, sem_m):
    _sc_body(tu, tm, users, movies, bias, out,
             uidx_v, midx_v, ublk_v, mblk_v, urow_v, mrow_v, bias_v, out_v,
             sem_u, sem_m)


def kernel(users, movies, user_table, item_table, W, b):
    users = users.astype(jnp.int32)
    movies = movies.astype(jnp.int32)
    ut_t = user_table.T
    it_t = item_table.T
    tu, tm = _tc_reduce(W, ut_t, ut_t, it_t, it_t)
    tu = tu.reshape(NROWS // 8, 8)
    tm = tm.reshape(NROWS // 8, 8)
    bias = jnp.full((16,), b[0], jnp.float32)
    out = _sc_gather(tu, tm, users, movies, bias)
    return out.reshape(BATCH, 1)
